# Initial kernel scaffold; baseline (speedup 1.0000x reference)
import jax
import jax.numpy as jnp
from jax.experimental import pallas as pl


def kernel(hs, U, neff, Ey):
    raise NotImplementedError("write your pallas kernel here")


# speedup vs baseline: 33.3535x; 33.3535x over previous
import jax
import jax.numpy as jnp
from jax.experimental import pallas as pl
from jax.experimental.pallas import tpu as pltpu

_N = 16
_MODES = 2
_OUT_RES = 32
_KNN = 2
_N0 = 1.0
_EY = 2 * (_KNN + 1) * _OUT_RES
_TOTAL = (_N + 2 * _KNN + 1) * _OUT_RES


def _body(u_ref, neff_ref, ey_ref, out_ref):
    i = pl.program_id(0)

    @pl.when(i == 0)
    def _():
        out_ref[...] = jnp.zeros_like(out_ref)

    r0 = i * _OUT_RES
    for j in range(_N):
        c = i * _N + j
        n0_ = neff_ref[c, 0]
        n1_ = neff_ref[c, 1]
        w0 = (n0_ * _N0 / (n0_ + _N0)) * u_ref[c, 0]
        w1 = (n1_ * _N0 / (n1_ + _N0)) * u_ref[c, 1]
        patch = ey_ref[0, j, 0] * w0 + ey_ref[0, j, 1] * w1
        out_ref[pl.ds(r0, _EY), j * _OUT_RES:j * _OUT_RES + _EY] += patch


def kernel(hs, U, neff, Ey):
    del hs
    en = pl.pallas_call(
        _body,
        grid=(_N,),
        in_specs=[
            pl.BlockSpec(memory_space=pltpu.SMEM),
            pl.BlockSpec(memory_space=pltpu.SMEM),
            pl.BlockSpec((1, _N, _MODES, _EY, _EY),
                         lambda i: (i, 0, 0, 0, 0)),
        ],
        out_specs=pl.BlockSpec((_TOTAL, _TOTAL), lambda i: (0, 0)),
        out_shape=jax.ShapeDtypeStruct((_TOTAL, _TOTAL), jnp.float32),
    )(U, neff, Ey.reshape(_N, _N, _MODES, _EY, _EY))
    return en.astype(jnp.complex64)
